# baseline (speedup 1.0000x reference)
import functools

import jax
import jax.numpy as jnp
import numpy as np
from jax.experimental import pallas as pl
from jax.experimental.pallas import tpu as pltpu

B, L, N, D = 16, 96, 32, 64
E, K = 4, 2
DFF = 128
NH = 4
DH = D // NH
PATCHES = (8, 6, 4, 2)
F = L // 2 + 1
BN = B * N


def _build_consts():
    l = np.arange(L)[:, None]
    f = np.arange(F)[None, :]
    ang = 2.0 * np.pi * l * f / L
    cos_m = np.cos(ang)
    sin_m = np.sin(ang)
    w = np.full(F, 2.0)
    w[0] = 1.0
    w[-1] = 1.0
    icos_m = (w[:, None] * np.cos(ang.T)) / L
    isin_m = (-w[:, None] * np.sin(ang.T)) / L
    trend = np.zeros((L, L))
    for kk in (3, 7, 11):
        for i in range(L):
            for j in range(i - (kk - 1) // 2, i + (kk - 1) // 2 + 1):
                trend[i, min(max(j, 0), L - 1)] += 1.0 / (3.0 * kk)
    amask = np.zeros((E, L, L), np.float32)
    for ei, p in enumerate(PATCHES):
        blk = (l // p) == (np.arange(L)[None, :] // p)
        amask[ei] = np.where(blk, 0.0, -1e30)
    return (cos_m.astype(np.float32), sin_m.astype(np.float32),
            icos_m.astype(np.float32), isin_m.astype(np.float32),
            trend.T.astype(np.float32), amask)


_COS, _SIN, _ICOS, _ISIN, _TREND_T, _AMASK = _build_consts()


def _gating_body(x0_ref, cos_ref, sin_ref, icos_ref, isin_ref, trt_ref,
                 slw_ref, slb_ref, wgt_ref, wgb_ref,
                 tki_ref, tkg_ref, bal_ref):
    x0 = x0_ref[...]
    xre = jnp.dot(x0, cos_ref[...], preferred_element_type=jnp.float32,
                  precision=jax.lax.Precision.HIGHEST)
    xim = -jnp.dot(x0, sin_ref[...], preferred_element_type=jnp.float32,
                   precision=jax.lax.Precision.HIGHEST)
    freq = jnp.sqrt(xre * xre + xim * xim)
    fidx = jax.lax.broadcasted_iota(jnp.int32, (BN, F), 1)
    freq = jnp.where(fidx == 0, 0.0, freq)
    m1 = jnp.max(freq, axis=1, keepdims=True)
    c1 = jnp.sum((freq >= m1).astype(jnp.float32), axis=1, keepdims=True)
    m2 = jnp.max(jnp.where(freq < m1, freq, -jnp.inf), axis=1, keepdims=True)
    c2 = jnp.sum((freq >= m2).astype(jnp.float32), axis=1, keepdims=True)
    m3 = jnp.max(jnp.where(freq < m2, freq, -jnp.inf), axis=1, keepdims=True)
    thresh = jnp.where(c1 >= 3.0, m1, jnp.where(c2 >= 3.0, m2, m3))
    mask = freq >= thresh
    xre_m = jnp.where(mask, xre, 0.0)
    xim_m = jnp.where(mask, xim, 0.0)
    season = (jnp.dot(xre_m, icos_ref[...], preferred_element_type=jnp.float32,
                      precision=jax.lax.Precision.HIGHEST)
              + jnp.dot(xim_m, isin_ref[...], preferred_element_type=jnp.float32,
                        precision=jax.lax.Precision.HIGHEST))
    tr = jnp.dot(x0, trt_ref[...], preferred_element_type=jnp.float32,
                 precision=jax.lax.Precision.HIGHEST)
    new = x0 + season + tr
    new3 = new.reshape(B, N, L)
    g = jnp.sum(new3 * slw_ref[...][None, :, :], axis=1) + slb_ref[...]
    logits = jnp.dot(g, wgt_ref[...], preferred_element_type=jnp.float32,
                     precision=jax.lax.Precision.HIGHEST) + wgb_ref[...]
    li = logits[:, :, None]
    lj = logits[:, None, :]
    ii = jax.lax.broadcasted_iota(jnp.int32, (B, E, E), 1)
    jj = jax.lax.broadcasted_iota(jnp.int32, (B, E, E), 2)
    beats = jnp.logical_or(lj > li, jnp.logical_and(lj == li, jj < ii))
    rank = jnp.sum(beats.astype(jnp.int32), axis=2)
    sel = rank < K
    mx = jnp.max(logits, axis=1, keepdims=True)
    ex = jnp.where(sel, jnp.exp(logits - mx), 0.0)
    gates = ex / jnp.sum(ex, axis=1, keepdims=True)
    eidx = jax.lax.broadcasted_iota(jnp.int32, (B, E), 1)
    top0 = jnp.sum(jnp.where(rank == 0, eidx, 0), axis=1, keepdims=True)
    top1 = jnp.sum(jnp.where(rank == 1, eidx, 0), axis=1, keepdims=True)
    tki_ref[...] = jnp.concatenate([top0, top1], axis=1)
    g0 = jnp.sum(jnp.where(rank == 0, gates, 0.0), axis=1, keepdims=True)
    g1 = jnp.sum(jnp.where(rank == 1, gates, 0.0), axis=1, keepdims=True)
    tkg_ref[...] = jnp.concatenate([g0, g1], axis=1)
    importance = jnp.sum(gates, axis=0, keepdims=True)
    load = jnp.sum((gates > 0).astype(jnp.float32), axis=0, keepdims=True)

    def cv2(v):
        mu = jnp.mean(v)
        var = jnp.sum((v - mu) ** 2) / (E - 1)
        return var / (mu * mu + 1e-10)

    bal_ref[...] = jnp.full((1, 1), (cv2(importance) + cv2(load)) * 1e-2,
                            jnp.float32)


def _layer_norm(x, g, b):
    mu = jnp.mean(x, axis=-1, keepdims=True)
    var = jnp.mean((x - mu) ** 2, axis=-1, keepdims=True)
    return (x - mu) * jax.lax.rsqrt(var + 1e-5) * g + b


_QK_SCALE = float(1.0 / np.sqrt(DH))


def _one_expert(xf, am, ipw, ipb, opw, opb, l1w, l1b, l2w, l2b, g1, b1, g2, b2):
    qkv = (jnp.dot(xf, ipw, preferred_element_type=jnp.float32)
           + ipb)
    ones_col = jnp.ones((N, L, 1), jnp.float32)
    outs = []
    for h in range(NH):
        qh = (qkv[:, h * DH:(h + 1) * DH] * _QK_SCALE).reshape(N, L, DH)
        kh = qkv[:, D + h * DH:D + (h + 1) * DH].reshape(N, L, DH)
        vh = qkv[:, 2 * D + h * DH:2 * D + (h + 1) * DH].reshape(N, L, DH)
        sc = jax.lax.dot_general(qh, kh, (((2,), (2,)), ((0,), (0,))),
                                 preferred_element_type=jnp.float32)
        sc = sc + am[None, :, :]
        p = jnp.exp(sc - jnp.max(sc, axis=-1, keepdims=True))
        ve = jnp.concatenate([vh, ones_col], axis=2)
        oe = jax.lax.dot_general(p, ve, (((2,), (1,)), ((0,), (0,))),
                                 preferred_element_type=jnp.float32)
        outs.append(oe[:, :, :DH] * (1.0 / oe[:, :, DH:DH + 1]))
    o = jnp.concatenate(outs, axis=-1).reshape(N * L, D)
    o = jnp.dot(o, opw, preferred_element_type=jnp.float32) + opb
    h1 = _layer_norm(xf + o, g1, b1)
    ff = jnp.dot(jax.nn.relu(
        jnp.dot(h1, l1w, preferred_element_type=jnp.float32)
        + l1b), l2w, preferred_element_type=jnp.float32) + l2b
    return _layer_norm(h1 + ff, g2, b2)


def _expert_body(tki_ref, tkg_ref, xt_ref, am0_ref, am1_ref,
                 ipw0, ipb0, opw0, opb0, l1w0, l1b0, l2w0, l2b0,
                 g10, b10, g20, b20,
                 ipw1, ipb1, opw1, opb1, l1w1, l1b1, l2w1, l2b1,
                 g11, b11, g21, b21, out_ref):
    b = pl.program_id(0)
    xb = xt_ref[0]
    xf = xb.reshape(N * L, D)
    h2a = _one_expert(xf, am0_ref[0], ipw0[0], ipb0[0], opw0[0], opb0[0],
                      l1w0[0], l1b0[0], l2w0[0], l2b0[0],
                      g10[0], b10[0], g20[0], b20[0])
    h2b = _one_expert(xf, am1_ref[0], ipw1[0], ipb1[0], opw1[0], opb1[0],
                      l1w1[0], l1b1[0], l2w1[0], l2b1[0],
                      g11[0], b11[0], g21[0], b21[0])
    comb = xf + tkg_ref[b, 0] * h2a + tkg_ref[b, 1] * h2b
    out_ref[0] = comb.reshape(N, L, D)


def kernel(x, sl_w, sl_b, wg_w, wg_b, inproj_w, inproj_b, outproj_w,
           outproj_b, lin1_w, lin1_b, lin2_w, lin2_b, n1_g, n1_b, n2_g, n2_b):
    x0t = x[:, :, :, 0].transpose(0, 2, 1).reshape(BN, L)
    tki, tkg, bal = pl.pallas_call(
        _gating_body,
        out_shape=(
            jax.ShapeDtypeStruct((B, K), jnp.int32),
            jax.ShapeDtypeStruct((B, K), jnp.float32),
            jax.ShapeDtypeStruct((1, 1), jnp.float32),
        ),
    )(x0t, jnp.asarray(_COS), jnp.asarray(_SIN), jnp.asarray(_ICOS),
      jnp.asarray(_ISIN), jnp.asarray(_TREND_T),
      sl_w.reshape(N, 1), sl_b.reshape(1, 1), wg_w.T, wg_b.reshape(1, E))

    xt = x.transpose(0, 2, 1, 3)

    def wspec(shp, kk):
        return pl.BlockSpec((1,) + shp,
                            lambda b, tki, tkg, _k=kk: (tki[b, _k], 0, 0))

    def expert_specs(kk):
        return [
            wspec((D, 3 * D), kk), wspec((1, 3 * D), kk),
            wspec((D, D), kk), wspec((1, D), kk),
            wspec((D, DFF), kk), wspec((1, DFF), kk),
            wspec((DFF, D), kk), wspec((1, D), kk),
            wspec((1, D), kk), wspec((1, D), kk),
            wspec((1, D), kk), wspec((1, D), kk),
        ]

    grid_spec = pltpu.PrefetchScalarGridSpec(
        num_scalar_prefetch=2,
        grid=(B,),
        in_specs=([pl.BlockSpec((1, N, L, D), lambda b, tki, tkg: (b, 0, 0, 0)),
                   wspec((L, L), 0), wspec((L, L), 1)]
                  + expert_specs(0) + expert_specs(1)),
        out_specs=pl.BlockSpec((1, N, L, D), lambda b, tki, tkg: (b, 0, 0, 0)),
    )
    wargs = (inproj_w.transpose(0, 2, 1), inproj_b.reshape(E, 1, 3 * D),
             outproj_w.transpose(0, 2, 1), outproj_b.reshape(E, 1, D),
             lin1_w.transpose(0, 2, 1), lin1_b.reshape(E, 1, DFF),
             lin2_w.transpose(0, 2, 1), lin2_b.reshape(E, 1, D),
             n1_g.reshape(E, 1, D), n1_b.reshape(E, 1, D),
             n2_g.reshape(E, 1, D), n2_b.reshape(E, 1, D))
    am = jnp.asarray(_AMASK)
    out_t = pl.pallas_call(
        _expert_body,
        grid_spec=grid_spec,
        out_shape=jax.ShapeDtypeStruct((B, N, L, D), jnp.float32),
    )(tki, tkg, xt, am, am, *wargs, *wargs)
    out = out_t.transpose(0, 2, 1, 3)
    return out, bal[0, 0]


# speedup vs baseline: 1.0383x; 1.0383x over previous
import functools

import jax
import jax.numpy as jnp
import numpy as np
from jax.experimental import pallas as pl
from jax.experimental.pallas import tpu as pltpu

B, L, N, D = 16, 96, 32, 64
E, K = 4, 2
DFF = 128
NH = 4
DH = D // NH
PATCHES = (8, 6, 4, 2)
F = L // 2 + 1
BN = B * N


def _build_consts():
    l = np.arange(L)[:, None]
    f = np.arange(F)[None, :]
    ang = 2.0 * np.pi * l * f / L
    cos_m = np.cos(ang)
    sin_m = np.sin(ang)
    w = np.full(F, 2.0)
    w[0] = 1.0
    w[-1] = 1.0
    icos_m = (w[:, None] * np.cos(ang.T)) / L
    isin_m = (-w[:, None] * np.sin(ang.T)) / L
    trend = np.zeros((L, L))
    for kk in (3, 7, 11):
        for i in range(L):
            for j in range(i - (kk - 1) // 2, i + (kk - 1) // 2 + 1):
                trend[i, min(max(j, 0), L - 1)] += 1.0 / (3.0 * kk)
    amask = np.zeros((E, L, L), np.float32)
    for ei, p in enumerate(PATCHES):
        blk = (l // p) == (np.arange(L)[None, :] // p)
        amask[ei] = np.where(blk, 0.0, -1e30)
    return (cos_m.astype(np.float32), sin_m.astype(np.float32),
            icos_m.astype(np.float32), isin_m.astype(np.float32),
            trend.T.astype(np.float32), amask)


_COS, _SIN, _ICOS, _ISIN, _TREND_T, _AMASK = _build_consts()


def _gating_body(x0_ref, cos_ref, sin_ref, icos_ref, isin_ref, trt_ref,
                 slw_ref, slb_ref, wgt_ref, wgb_ref,
                 tki_ref, tkg_ref, bal_ref):
    x0 = x0_ref[...]
    xre = jnp.dot(x0, cos_ref[...], preferred_element_type=jnp.float32,
                  precision=jax.lax.Precision.HIGHEST)
    xim = -jnp.dot(x0, sin_ref[...], preferred_element_type=jnp.float32,
                   precision=jax.lax.Precision.HIGHEST)
    freq = jnp.sqrt(xre * xre + xim * xim)
    fidx = jax.lax.broadcasted_iota(jnp.int32, (BN, F), 1)
    freq = jnp.where(fidx == 0, 0.0, freq)
    m1 = jnp.max(freq, axis=1, keepdims=True)
    c1 = jnp.sum((freq >= m1).astype(jnp.float32), axis=1, keepdims=True)
    m2 = jnp.max(jnp.where(freq < m1, freq, -jnp.inf), axis=1, keepdims=True)
    c2 = jnp.sum((freq >= m2).astype(jnp.float32), axis=1, keepdims=True)
    m3 = jnp.max(jnp.where(freq < m2, freq, -jnp.inf), axis=1, keepdims=True)
    thresh = jnp.where(c1 >= 3.0, m1, jnp.where(c2 >= 3.0, m2, m3))
    mask = freq >= thresh
    xre_m = jnp.where(mask, xre, 0.0)
    xim_m = jnp.where(mask, xim, 0.0)
    season = (jnp.dot(xre_m, icos_ref[...], preferred_element_type=jnp.float32,
                      precision=jax.lax.Precision.HIGHEST)
              + jnp.dot(xim_m, isin_ref[...], preferred_element_type=jnp.float32,
                        precision=jax.lax.Precision.HIGHEST))
    tr = jnp.dot(x0, trt_ref[...], preferred_element_type=jnp.float32,
                 precision=jax.lax.Precision.HIGHEST)
    new = x0 + season + tr
    new3 = new.reshape(B, N, L)
    g = jnp.sum(new3 * slw_ref[...][None, :, :], axis=1) + slb_ref[...]
    logits = jnp.dot(g, wgt_ref[...], preferred_element_type=jnp.float32,
                     precision=jax.lax.Precision.HIGHEST) + wgb_ref[...]
    li = logits[:, :, None]
    lj = logits[:, None, :]
    ii = jax.lax.broadcasted_iota(jnp.int32, (B, E, E), 1)
    jj = jax.lax.broadcasted_iota(jnp.int32, (B, E, E), 2)
    beats = jnp.logical_or(lj > li, jnp.logical_and(lj == li, jj < ii))
    rank = jnp.sum(beats.astype(jnp.int32), axis=2)
    sel = rank < K
    mx = jnp.max(logits, axis=1, keepdims=True)
    ex = jnp.where(sel, jnp.exp(logits - mx), 0.0)
    gates = ex / jnp.sum(ex, axis=1, keepdims=True)
    eidx = jax.lax.broadcasted_iota(jnp.int32, (B, E), 1)
    top0 = jnp.sum(jnp.where(rank == 0, eidx, 0), axis=1, keepdims=True)
    top1 = jnp.sum(jnp.where(rank == 1, eidx, 0), axis=1, keepdims=True)
    tki_ref[...] = jnp.concatenate([top0, top1], axis=1)
    g0 = jnp.sum(jnp.where(rank == 0, gates, 0.0), axis=1, keepdims=True)
    g1 = jnp.sum(jnp.where(rank == 1, gates, 0.0), axis=1, keepdims=True)
    tkg_ref[...] = jnp.concatenate([g0, g1], axis=1)
    importance = jnp.sum(gates, axis=0, keepdims=True)
    load = jnp.sum((gates > 0).astype(jnp.float32), axis=0, keepdims=True)

    def cv2(v):
        mu = jnp.mean(v)
        var = jnp.sum((v - mu) ** 2) / (E - 1)
        return var / (mu * mu + 1e-10)

    bal_ref[...] = jnp.full((1, 1), (cv2(importance) + cv2(load)) * 1e-2,
                            jnp.float32)


def _layer_norm(x, g, b):
    mu = jnp.mean(x, axis=-1, keepdims=True)
    var = jnp.mean((x - mu) ** 2, axis=-1, keepdims=True)
    return (x - mu) * jax.lax.rsqrt(var + 1e-5) * g + b


_QK_SCALE = float(1.0 / np.sqrt(DH))


def _one_expert(xf, am, ipw, ipb, opw, opb, l1w, l1b, l2w, l2b, g1, b1, g2, b2):
    qkv = (jnp.dot(xf, ipw, preferred_element_type=jnp.float32)
           + ipb)
    outs = []
    for h in range(NH):
        qh = (qkv[:, h * DH:(h + 1) * DH] * _QK_SCALE).reshape(N, L, DH)
        kh = qkv[:, D + h * DH:D + (h + 1) * DH].reshape(N, L, DH)
        vh = qkv[:, 2 * D + h * DH:2 * D + (h + 1) * DH].reshape(N, L, DH)
        sc = jax.lax.dot_general(qh, kh, (((2,), (2,)), ((0,), (0,))),
                                 preferred_element_type=jnp.float32)
        sc = sc + am[None, :, :]
        p = jnp.exp(sc - jnp.max(sc, axis=-1, keepdims=True))
        pinv = 1.0 / jnp.sum(p, axis=-1, keepdims=True)
        oh = jax.lax.dot_general(p, vh, (((2,), (1,)), ((0,), (0,))),
                                 preferred_element_type=jnp.float32)
        outs.append(oh * pinv)
    o = jnp.concatenate(outs, axis=-1).reshape(N * L, D)
    o = jnp.dot(o, opw, preferred_element_type=jnp.float32) + opb
    h1 = _layer_norm(xf + o, g1, b1)
    ff = jnp.dot(jax.nn.relu(
        jnp.dot(h1, l1w, preferred_element_type=jnp.float32)
        + l1b), l2w, preferred_element_type=jnp.float32) + l2b
    return _layer_norm(h1 + ff, g2, b2)


def _expert_body(tki_ref, tkg_ref, xt_ref, am0_ref, am1_ref,
                 ipw0, ipb0, opw0, opb0, l1w0, l1b0, l2w0, l2b0,
                 g10, b10, g20, b20,
                 ipw1, ipb1, opw1, opb1, l1w1, l1b1, l2w1, l2b1,
                 g11, b11, g21, b21, out_ref):
    b = pl.program_id(0)
    xb = xt_ref[0]
    xf = xb.reshape(N * L, D)
    h2a = _one_expert(xf, am0_ref[0], ipw0[0], ipb0[0], opw0[0], opb0[0],
                      l1w0[0], l1b0[0], l2w0[0], l2b0[0],
                      g10[0], b10[0], g20[0], b20[0])
    h2b = _one_expert(xf, am1_ref[0], ipw1[0], ipb1[0], opw1[0], opb1[0],
                      l1w1[0], l1b1[0], l2w1[0], l2b1[0],
                      g11[0], b11[0], g21[0], b21[0])
    comb = xf + tkg_ref[b, 0] * h2a + tkg_ref[b, 1] * h2b
    out_ref[0] = comb.reshape(N, L, D)


def kernel(x, sl_w, sl_b, wg_w, wg_b, inproj_w, inproj_b, outproj_w,
           outproj_b, lin1_w, lin1_b, lin2_w, lin2_b, n1_g, n1_b, n2_g, n2_b):
    x0t = x[:, :, :, 0].transpose(0, 2, 1).reshape(BN, L)
    tki, tkg, bal = pl.pallas_call(
        _gating_body,
        out_shape=(
            jax.ShapeDtypeStruct((B, K), jnp.int32),
            jax.ShapeDtypeStruct((B, K), jnp.float32),
            jax.ShapeDtypeStruct((1, 1), jnp.float32),
        ),
    )(x0t, jnp.asarray(_COS), jnp.asarray(_SIN), jnp.asarray(_ICOS),
      jnp.asarray(_ISIN), jnp.asarray(_TREND_T),
      sl_w.reshape(N, 1), sl_b.reshape(1, 1), wg_w.T, wg_b.reshape(1, E))

    xt = x.transpose(0, 2, 1, 3)

    def wspec(shp, kk):
        return pl.BlockSpec((1,) + shp,
                            lambda b, tki, tkg, _k=kk: (tki[b, _k], 0, 0))

    def expert_specs(kk):
        return [
            wspec((D, 3 * D), kk), wspec((1, 3 * D), kk),
            wspec((D, D), kk), wspec((1, D), kk),
            wspec((D, DFF), kk), wspec((1, DFF), kk),
            wspec((DFF, D), kk), wspec((1, D), kk),
            wspec((1, D), kk), wspec((1, D), kk),
            wspec((1, D), kk), wspec((1, D), kk),
        ]

    grid_spec = pltpu.PrefetchScalarGridSpec(
        num_scalar_prefetch=2,
        grid=(B,),
        in_specs=([pl.BlockSpec((1, N, L, D), lambda b, tki, tkg: (b, 0, 0, 0)),
                   wspec((L, L), 0), wspec((L, L), 1)]
                  + expert_specs(0) + expert_specs(1)),
        out_specs=pl.BlockSpec((1, N, L, D), lambda b, tki, tkg: (b, 0, 0, 0)),
    )
    wargs = (inproj_w.transpose(0, 2, 1), inproj_b.reshape(E, 1, 3 * D),
             outproj_w.transpose(0, 2, 1), outproj_b.reshape(E, 1, D),
             lin1_w.transpose(0, 2, 1), lin1_b.reshape(E, 1, DFF),
             lin2_w.transpose(0, 2, 1), lin2_b.reshape(E, 1, D),
             n1_g.reshape(E, 1, D), n1_b.reshape(E, 1, D),
             n2_g.reshape(E, 1, D), n2_b.reshape(E, 1, D))
    am = jnp.asarray(_AMASK)
    out_t = pl.pallas_call(
        _expert_body,
        grid_spec=grid_spec,
        out_shape=jax.ShapeDtypeStruct((B, N, L, D), jnp.float32),
    )(tki, tkg, xt, am, am, *wargs, *wargs)
    out = out_t.transpose(0, 2, 1, 3)
    return out, bal[0, 0]
